# baseline (speedup 1.0000x reference)
import jax
import jax.numpy as jnp
from jax import lax
from jax.experimental import pallas as pl
from jax.experimental.pallas import tpu as pltpu
from jax.experimental.pallas import tpu_sc as plsc

VOCAB = 1000000
B = 16384
M = 100

NC = 2
NS = 16
NW = NC * NS
BPW = B // NW
TBL_PAD = VOCAB + 16
CHUNK = 62504
CM = 10
NCHUNK = M // CM
CW = CM * BPW
SR = CW


def _sc_kernel(idx_hbm, w_hbm, out_hbm, idx_a, idx_b, vals_a, vals_b,
               out_v, zrow_v, table_sh, isem, gsem):
    c = lax.axis_index("c")
    s = lax.axis_index("s")
    wid = c * NS + s
    b0 = wid * BPW

    ibufs = (idx_a, idx_b)
    vbufs = (vals_a, vals_b)

    def stage_idx(k, buf):
        return [
            pltpu.async_copy(
                idx_hbm.at[pl.ds((k * CM + i) * B + b0, BPW)],
                buf.at[pl.ds(i * BPW, BPW)], isem)
            for i in range(CM)
        ]

    descs = {0: stage_idx(0, idx_a)}

    base = s * CHUNK

    @pl.when(s < NS - 1)
    def _():
        for off in range(0, CHUNK, SR):
            n = min(SR, CHUNK - off)
            pltpu.sync_copy(w_hbm.at[pl.ds(base + off, n)],
                            vals_a.at[pl.ds(0, n)])
            pltpu.sync_copy(vals_a.at[pl.ds(0, n)],
                            table_sh.at[pl.ds(base + off, n)])

    @pl.when(s == NS - 1)
    def _():
        last = (NS - 1) * CHUNK
        rem = VOCAB - last
        for off in range(0, rem, SR):
            n = min(SR, rem - off)
            pltpu.sync_copy(w_hbm.at[pl.ds(last + off, n)],
                            vals_a.at[pl.ds(0, n)])
            pltpu.sync_copy(vals_a.at[pl.ds(0, n)],
                            table_sh.at[pl.ds(last + off, n)])

    @pl.when(s == 0)
    def _():
        zrow_v[...] = jnp.zeros((16,), jnp.float32)
        pltpu.sync_copy(zrow_v, table_sh.at[pl.ds(VOCAB, 16)])

    plsc.subcore_barrier()

    def gather(k):
        return pltpu.async_copy(
            table_sh.at[ibufs[k % 2]], vbufs[k % 2], gsem)

    for d in descs[0]:
        d.wait()
    cps = {0: gather(0)}
    descs[1] = stage_idx(1, idx_b)

    for k in range(NCHUNK):
        buf = vbufs[k % 2]
        cps[k].wait()
        if k + 1 < NCHUNK:
            for d in descs[k + 1]:
                d.wait()
            cps[k + 1] = gather(k + 1)
            if k + 2 < NCHUNK:
                descs[k + 2] = stage_idx(k + 2, ibufs[k % 2])

        def body(g, _):
            gbase = pl.multiple_of(g * 16, 16)
            acc = buf[pl.ds(gbase, 16)]
            for mi in range(1, CM):
                acc = acc + buf[pl.ds(mi * BPW + gbase, 16)]
            if k == 0:
                out_v[pl.ds(gbase, 16)] = acc
            else:
                out_v[pl.ds(gbase, 16)] = out_v[pl.ds(gbase, 16)] + acc
            return 0

        lax.fori_loop(0, BPW // 16, body, 0)

    pltpu.sync_copy(out_v, out_hbm.at[pl.ds(b0, BPW)])


@jax.jit
def kernel(inputs, W):
    idx_mm = inputs.T.reshape(M * B)
    w_flat = W[jnp.arange(VOCAB + 1), 0]

    run = pl.kernel(
        _sc_kernel,
        out_type=jax.ShapeDtypeStruct((B,), jnp.float32),
        mesh=plsc.VectorSubcoreMesh(core_axis_name="c", subcore_axis_name="s",
                                    num_cores=NC, num_subcores=NS),
        scratch_types=[
            pltpu.VMEM((CW,), jnp.int32),
            pltpu.VMEM((CW,), jnp.int32),
            pltpu.VMEM((CW,), jnp.float32),
            pltpu.VMEM((CW,), jnp.float32),
            pltpu.VMEM((BPW,), jnp.float32),
            pltpu.VMEM((16,), jnp.float32),
            pltpu.VMEM_SHARED((TBL_PAD,), jnp.float32),
            pltpu.SemaphoreType.DMA,
            pltpu.SemaphoreType.DMA,
        ],
    )
    out = run(idx_mm, w_flat)
    return out.reshape(B, 1)


# speedup vs baseline: 151.1724x; 151.1724x over previous
import jax
import jax.numpy as jnp
from jax import lax
from jax.experimental import pallas as pl
from jax.experimental.pallas import tpu as pltpu
from jax.experimental.pallas import tpu_sc as plsc

VOCAB = 1000000
B = 16384
M = 100

NC = 2
NS = 16
NW = NC * NS
BPW = B // NW
TBL_PAD = VOCAB + 16
CHUNK = 62504
CM = 10
NCHUNK = M // CM
CW = CM * BPW
SR = CW


def _sc_kernel(idx_hbm, w_hbm, out_hbm, idx_a, idx_b, vals_a, vals_b,
               out_v, zrow_v, table_sh, isem, gsem):
    c = lax.axis_index("c")
    s = lax.axis_index("s")
    wid = c * NS + s
    b0 = wid * BPW

    ibufs = (idx_a, idx_b)
    vbufs = (vals_a, vals_b)

    def stage_idx(k, buf):
        return [
            pltpu.async_copy(
                idx_hbm.at[pl.ds((k * CM + i) * B + b0, BPW)],
                buf.at[pl.ds(i * BPW, BPW)], isem)
            for i in range(CM)
        ]

    descs = {0: stage_idx(0, idx_a)}

    base = s * CHUNK

    @pl.when(s < NS - 1)
    def _():
        for off in range(0, CHUNK, SR):
            n = min(SR, CHUNK - off)
            pltpu.sync_copy(w_hbm.at[pl.ds(base + off, n)],
                            vals_a.at[pl.ds(0, n)])
            pltpu.sync_copy(vals_a.at[pl.ds(0, n)],
                            table_sh.at[pl.ds(base + off, n)])

    @pl.when(s == NS - 1)
    def _():
        last = (NS - 1) * CHUNK
        rem = VOCAB - last
        for off in range(0, rem, SR):
            n = min(SR, rem - off)
            pltpu.sync_copy(w_hbm.at[pl.ds(last + off, n)],
                            vals_a.at[pl.ds(0, n)])
            pltpu.sync_copy(vals_a.at[pl.ds(0, n)],
                            table_sh.at[pl.ds(last + off, n)])

    @pl.when(s == 0)
    def _():
        zrow_v[...] = jnp.zeros((16,), jnp.float32)
        pltpu.sync_copy(zrow_v, table_sh.at[pl.ds(VOCAB, 16)])

    plsc.subcore_barrier()

    def gather(k):
        return pltpu.async_copy(
            table_sh.at[ibufs[k % 2]], vbufs[k % 2], gsem)

    for d in descs[0]:
        d.wait()
    cps = {0: gather(0)}
    descs[1] = stage_idx(1, idx_b)

    for k in range(NCHUNK):
        buf = vbufs[k % 2]
        cps[k].wait()
        if k + 1 < NCHUNK:
            for d in descs[k + 1]:
                d.wait()
            cps[k + 1] = gather(k + 1)
            if k + 2 < NCHUNK:
                descs[k + 2] = stage_idx(k + 2, ibufs[k % 2])

        def body(g, _):
            gbase = pl.multiple_of(g * 16, 16)
            acc = buf[pl.ds(gbase, 16)]
            for mi in range(1, CM):
                acc = acc + buf[pl.ds(mi * BPW + gbase, 16)]
            if k == 0:
                out_v[pl.ds(gbase, 16)] = acc
            else:
                out_v[pl.ds(gbase, 16)] = out_v[pl.ds(gbase, 16)] + acc
            return 0

        lax.fori_loop(0, BPW // 16, body, 0)

    pltpu.sync_copy(out_v, out_hbm.at[pl.ds(b0, BPW)])


@jax.jit
def kernel(inputs, W):
    idx_mm = inputs.T.reshape(M * B)
    w_flat = jnp.pad(W, ((0, 447), (0, 0))).reshape(7816, 128).reshape(1000448)

    run = pl.kernel(
        _sc_kernel,
        out_type=jax.ShapeDtypeStruct((B,), jnp.float32),
        mesh=plsc.VectorSubcoreMesh(core_axis_name="c", subcore_axis_name="s",
                                    num_cores=NC, num_subcores=NS),
        scratch_types=[
            pltpu.VMEM((CW,), jnp.int32),
            pltpu.VMEM((CW,), jnp.int32),
            pltpu.VMEM((CW,), jnp.float32),
            pltpu.VMEM((CW,), jnp.float32),
            pltpu.VMEM((BPW,), jnp.float32),
            pltpu.VMEM((16,), jnp.float32),
            pltpu.VMEM_SHARED((TBL_PAD,), jnp.float32),
            pltpu.SemaphoreType.DMA,
            pltpu.SemaphoreType.DMA,
        ],
    )
    out = run(idx_mm, w_flat)
    return out.reshape(B, 1)
